# baseline (speedup 1.0000x reference)
import functools

import jax
import jax.numpy as jnp
from jax import lax
from jax.experimental import pallas as pl
from jax.experimental.pallas import tpu as pltpu
from jax.experimental.pallas import tpu_sc as plsc

N_TOKENS = 16384
N_CODES = 8192
DIM = 64

BR = 256
NB = N_TOKENS // BR


def _argmin_body(x_ref, cb_ref, idx_ref):
    x = x_ref[...]
    cb = cb_ref[...]
    norm = jnp.sqrt(jnp.sum(x * x, axis=-1, keepdims=True))
    xn = x / jnp.maximum(norm, 1e-12)
    mm = lax.dot_general(xn.astype(jnp.bfloat16), cb.astype(jnp.bfloat16),
                         (((1,), (1,)), ((), ())),
                         preferred_element_type=jnp.float32)
    xsq = jnp.sum(xn * xn, axis=-1, keepdims=True)
    csq = jnp.sum(cb * cb, axis=-1)[None, :]
    dist = (xsq - 2.0 * mm) + csq
    acc = jnp.full((BR,), jnp.inf, jnp.float32)
    ai = jnp.zeros((BR,), jnp.int32)
    for lo, hi in ((0, 4096), (4096, N_CODES)):
        sub = dist[:, lo:hi]
        m = jnp.min(sub, axis=1)
        iota = lax.broadcasted_iota(jnp.int32, sub.shape, 1) + lo
        gi = jnp.min(jnp.where(sub == m[:, None], iota, N_CODES), axis=1)
        take = m < acc.astype(jnp.bfloat16).astype(jnp.float32)
        ai = jnp.where(take, gi, ai)
        acc = jnp.where(take, m, acc)
    idx_ref[...] = ai[None, None, :]


def _compute_code_idx(x, codebook):
    out = pl.pallas_call(
        _argmin_body,
        grid=(NB,),
        in_specs=[
            pl.BlockSpec((BR, DIM), lambda i: (i, 0)),
            pl.BlockSpec((N_CODES, DIM), lambda i: (0, 0)),
        ],
        out_specs=pl.BlockSpec((1, 1, BR), lambda i: (i, 0, 0)),
        out_shape=jax.ShapeDtypeStruct((NB, 1, BR), jnp.int32),
    )(x, codebook)
    return out.reshape(N_TOKENS)


def _make_sc_gather():
    info = plsc.get_sparse_core_info()
    nc, ns = info.num_cores, info.num_subcores
    nw = nc * ns
    b_per_w = N_TOKENS // nw
    mesh = plsc.VectorSubcoreMesh(core_axis_name="c", subcore_axis_name="s")

    @functools.partial(
        pl.kernel, mesh=mesh,
        out_type=jax.ShapeDtypeStruct((N_TOKENS, DIM), jnp.float32),
        scratch_types=[
            pltpu.VMEM((b_per_w,), jnp.int32),
            pltpu.VMEM((b_per_w, DIM), jnp.float32),
            pltpu.SemaphoreType.DMA,
        ],
        compiler_params=pltpu.CompilerParams(use_tc_tiling_on_sc=False),
    )
    def gather(table_hbm, idx_hbm, out_hbm, idx_v, rows_v, sem):
        wid = lax.axis_index("s") * nc + lax.axis_index("c")
        base = wid * b_per_w
        pltpu.sync_copy(idx_hbm.at[pl.ds(base, b_per_w)], idx_v)
        pltpu.async_copy(table_hbm.at[idx_v], rows_v, sem).wait()
        pltpu.sync_copy(rows_v, out_hbm.at[pl.ds(base, b_per_w)])

    return gather


_sc_gather_cache = []


def kernel(x, codebook):
    code_idx = _compute_code_idx(x, codebook)
    if not _sc_gather_cache:
        _sc_gather_cache.append(_make_sc_gather())
    x_d = _sc_gather_cache[0](codebook, code_idx)
    return (x_d, code_idx)


# speedup vs baseline: 1.1602x; 1.1602x over previous
import functools

import jax
import jax.numpy as jnp
from jax import lax
from jax.experimental import pallas as pl
from jax.experimental.pallas import tpu as pltpu
from jax.experimental.pallas import tpu_sc as plsc

N_TOKENS = 16384
N_CODES = 8192
DIM = 64

BR = 256
NB = N_TOKENS // BR


def _argmin_body(x_ref, cb_ref, csq_ref, idx_ref):
    x = x_ref[...]
    cb = cb_ref[...]
    norm = jnp.sqrt(jnp.sum(x * x, axis=-1, keepdims=True))
    xn = x / jnp.maximum(norm, 1e-12)
    mm = lax.dot_general(xn.astype(jnp.bfloat16), cb,
                         (((1,), (1,)), ((), ())),
                         preferred_element_type=jnp.float32)
    xsq = jnp.sum(xn * xn, axis=-1, keepdims=True)
    csq = csq_ref[...]
    dist = (xsq - 2.0 * mm) + csq
    acc = jnp.full((BR,), jnp.inf, jnp.float32)
    ai = jnp.zeros((BR,), jnp.int32)
    for lo, hi in ((0, 4096), (4096, N_CODES)):
        sub = dist[:, lo:hi]
        m = jnp.min(sub, axis=1)
        iota = lax.broadcasted_iota(jnp.int32, sub.shape, 1) + lo
        gi = jnp.min(jnp.where(sub == m[:, None], iota, N_CODES), axis=1)
        take = m < acc.astype(jnp.bfloat16).astype(jnp.float32)
        ai = jnp.where(take, gi, ai)
        acc = jnp.where(take, m, acc)
    idx_ref[...] = ai[None, None, :]


def _compute_code_idx(x, codebook):
    cb_bf = codebook.astype(jnp.bfloat16)
    csq = jnp.sum(codebook * codebook, axis=-1)[None, :]
    out = pl.pallas_call(
        _argmin_body,
        grid=(NB,),
        in_specs=[
            pl.BlockSpec((BR, DIM), lambda i: (i, 0)),
            pl.BlockSpec((N_CODES, DIM), lambda i: (0, 0)),
            pl.BlockSpec((1, N_CODES), lambda i: (0, 0)),
        ],
        out_specs=pl.BlockSpec((1, 1, BR), lambda i: (i, 0, 0)),
        out_shape=jax.ShapeDtypeStruct((NB, 1, BR), jnp.int32),
        compiler_params=pltpu.CompilerParams(
            dimension_semantics=("parallel",)),
    )(x, cb_bf, csq)
    return out.reshape(N_TOKENS)


def _make_sc_gather():
    info = plsc.get_sparse_core_info()
    nc, ns = info.num_cores, info.num_subcores
    nw = nc * ns
    b_per_w = N_TOKENS // nw
    mesh = plsc.VectorSubcoreMesh(core_axis_name="c", subcore_axis_name="s")

    @functools.partial(
        pl.kernel, mesh=mesh,
        out_type=jax.ShapeDtypeStruct((N_TOKENS, DIM), jnp.float32),
        scratch_types=[
            pltpu.VMEM((b_per_w,), jnp.int32),
            pltpu.VMEM((b_per_w, DIM), jnp.float32),
            pltpu.SemaphoreType.DMA,
        ],
        compiler_params=pltpu.CompilerParams(use_tc_tiling_on_sc=False),
    )
    def gather(table_hbm, idx_hbm, out_hbm, idx_v, rows_v, sem):
        wid = lax.axis_index("s") * nc + lax.axis_index("c")
        base = wid * b_per_w
        pltpu.sync_copy(idx_hbm.at[pl.ds(base, b_per_w)], idx_v)
        pltpu.async_copy(table_hbm.at[idx_v], rows_v, sem).wait()
        pltpu.sync_copy(rows_v, out_hbm.at[pl.ds(base, b_per_w)])

    return gather


_sc_gather_cache = []


def kernel(x, codebook):
    code_idx = _compute_code_idx(x, codebook)
    if not _sc_gather_cache:
        _sc_gather_cache.append(_make_sc_gather())
    x_d = _sc_gather_cache[0](codebook, code_idx)
    return (x_d, code_idx)
